# baseline (speedup 1.0000x reference)
import functools

import jax
import jax.numpy as jnp
from jax import lax
from jax.experimental import pallas as pl
from jax.experimental.pallas import tpu as pltpu
from jax.experimental.pallas import tpu_sc as plsc

SDIM = 128
EDIM = 16
KEEP = 145
TW = 160
SCATW = 144
CH = 128
NWORK = 32


def _posmask(w):
    io = lax.broadcasted_iota(jnp.int32, (1, w), 1)
    return jnp.where((io >= 4) & (io < 7), 1.0, 0.0).astype(jnp.float32)


def _cntmask(w):
    io = lax.broadcasted_iota(jnp.int32, (1, w), 1)
    return jnp.where(io == (w - 1), 1.0, 0.0).astype(jnp.float32)


def _sc_gather2(tab_a, tab_b, idx_a, idx_b, n_out):
    wa = tab_a.shape[1]
    wb = tab_b.shape[1]
    per_w = n_out // NWORK
    nch = per_w // CH
    mesh = plsc.VectorSubcoreMesh(core_axis_name="c", subcore_axis_name="s", num_cores=2, num_subcores=16)

    @functools.partial(
        pl.kernel,
        out_type=(jax.ShapeDtypeStruct((n_out, wa), jnp.float32),
                  jax.ShapeDtypeStruct((n_out, wb), jnp.float32)),
        mesh=mesh,
        compiler_params=pltpu.CompilerParams(use_tc_tiling_on_sc=False),
        scratch_types=[
            pltpu.VMEM((per_w,), jnp.int32),
            pltpu.VMEM((per_w,), jnp.int32),
            pltpu.VMEM((2, CH, wa), jnp.float32),
            pltpu.VMEM((2, CH, wb), jnp.float32),
            pltpu.SemaphoreType.DMA,
            pltpu.SemaphoreType.DMA,
            pltpu.SemaphoreType.DMA,
            pltpu.SemaphoreType.DMA,
            pltpu.SemaphoreType.DMA,
            pltpu.SemaphoreType.DMA,
            pltpu.SemaphoreType.DMA,
            pltpu.SemaphoreType.DMA,
        ],
    )
    def k(ta_hbm, tb_hbm, ia_hbm, ib_hbm, ga_hbm, gb_hbm,
          ia_v, ib_v, ra_v, rb_v,
          sga0, sga1, sgb0, sgb1, swa0, swa1, swb0, swb1):
        wid = lax.axis_index("s") * 2 + lax.axis_index("c")
        base = wid * per_w
        pltpu.sync_copy(ia_hbm.at[pl.ds(base, per_w)], ia_v)
        pltpu.sync_copy(ib_hbm.at[pl.ds(base, per_w)], ib_v)
        sga = (sga0, sga1)
        sgb = (sgb0, sgb1)
        swa = (swa0, swa1)
        swb = (swb0, swb1)

        def start_gather(j, b):
            ca = pltpu.async_copy(ta_hbm.at[ia_v.at[pl.ds(j * CH, CH)]],
                                  ra_v.at[b], sga[b])
            cb = pltpu.async_copy(tb_hbm.at[ib_v.at[pl.ds(j * CH, CH)]],
                                  rb_v.at[b], sgb[b])
            return ca, cb

        def start_write(j, b):
            pltpu.async_copy(ra_v.at[b], ga_hbm.at[pl.ds(base + j * CH, CH)],
                             swa[b])
            pltpu.async_copy(rb_v.at[b], gb_hbm.at[pl.ds(base + j * CH, CH)],
                             swb[b])

        def drain_write(b):
            pltpu.make_async_copy(ra_v.at[b], ga_hbm.at[pl.ds(base, CH)],
                                  swa[b]).wait()
            pltpu.make_async_copy(rb_v.at[b], gb_hbm.at[pl.ds(base, CH)],
                                  swb[b]).wait()

        def group(g, carry):
            j0 = g * 2
            j1 = j0 + 1

            @pl.when(g > 0)
            def _():
                drain_write(0)
                drain_write(1)

            ca0, cb0 = start_gather(j0, 0)
            ca1, cb1 = start_gather(j1, 1)
            ca0.wait()
            cb0.wait()
            start_write(j0, 0)
            ca1.wait()
            cb1.wait()
            start_write(j1, 1)
            return carry

        lax.fori_loop(0, nch // 2, group, 0)
        if nch % 2:
            j = nch - 1
            drain_write(0)
            ca, cb = start_gather(j, 0)
            ca.wait()
            cb.wait()
            start_write(j, 0)
            drain_write(0)
            drain_write(1)
        else:
            drain_write(0)
            drain_write(1)

    return k(tab_a, tab_b, idx_a, idx_b)


def _sc_gather_sum(tab_a, tab_b, pos_tab, idx_a, idx_b, n_out):
    w = tab_a.shape[1]
    per_w = n_out // NWORK
    nch = per_w // CH
    assert nch % 2 == 0
    mesh = plsc.VectorSubcoreMesh(core_axis_name="c", subcore_axis_name="s", num_cores=2, num_subcores=16)
    with_pos = pos_tab is not None

    out_type = [jax.ShapeDtypeStruct((n_out, w), jnp.float32)]
    scratch = [
        pltpu.VMEM((per_w,), jnp.int32),
        pltpu.VMEM((per_w,), jnp.int32),
        pltpu.VMEM((2, CH, w), jnp.float32),
    ]
    if with_pos:
        out_type.append(jax.ShapeDtypeStruct((n_out, 16), jnp.float32))
        scratch.append(pltpu.VMEM((2, CH, 16), jnp.float32))
    scratch += [pltpu.SemaphoreType.DMA] * (10 if with_pos else 6)

    @functools.partial(
        pl.kernel,
        out_type=tuple(out_type),
        mesh=mesh,
        compiler_params=pltpu.CompilerParams(use_tc_tiling_on_sc=False),
        scratch_types=scratch,
    )
    def k(*refs):
        if with_pos:
            (ta_hbm, tb_hbm, pt_hbm, ia_hbm, ib_hbm, gs_hbm, gp_hbm,
             ia_v, ib_v, ra_v, rp_v,
             sga0, sga1, sgb0, sgb1, sgp0, sgp1, swa0, swa1, swp0, swp1) = refs
            sgp = (sgp0, sgp1)
            swp = (swp0, swp1)
        else:
            (ta_hbm, tb_hbm, ia_hbm, ib_hbm, gs_hbm,
             ia_v, ib_v, ra_v,
             sga0, sga1, sgb0, sgb1, swa0, swa1) = refs
        sga = (sga0, sga1)
        sgb = (sgb0, sgb1)
        swa = (swa0, swa1)
        wid = lax.axis_index("s") * 2 + lax.axis_index("c")
        base = wid * per_w
        pltpu.sync_copy(ia_hbm.at[pl.ds(base, per_w)], ia_v)
        pltpu.sync_copy(ib_hbm.at[pl.ds(base, per_w)], ib_v)

        def start_base(j, b):
            ca = pltpu.async_copy(ta_hbm.at[ia_v.at[pl.ds(j * CH, CH)]],
                                  ra_v.at[b], sga[b])
            cp = None
            if with_pos:
                cp = pltpu.async_copy(pt_hbm.at[ia_v.at[pl.ds(j * CH, CH)]],
                                      rp_v.at[b], sgp[b])
            return ca, cp

        def start_add(j, b):
            return pltpu.async_copy(tb_hbm.at[ib_v.at[pl.ds(j * CH, CH)]],
                                    ra_v.at[b], sgb[b], add=True)

        def chunk_back(j, b, cb, cp):
            cb.wait()
            pltpu.async_copy(ra_v.at[b], gs_hbm.at[pl.ds(base + j * CH, CH)],
                             swa[b])
            if with_pos:
                cp.wait()
                pltpu.async_copy(rp_v.at[b],
                                 gp_hbm.at[pl.ds(base + j * CH, CH)], swp[b])

        def drain_write(b):
            pltpu.make_async_copy(ra_v.at[b], gs_hbm.at[pl.ds(base, CH)],
                                  swa[b]).wait()
            if with_pos:
                pltpu.make_async_copy(rp_v.at[b], gp_hbm.at[pl.ds(base, CH)],
                                      swp[b]).wait()

        def group(g, carry):
            j0 = g * 2
            j1 = j0 + 1

            @pl.when(g > 0)
            def _():
                drain_write(0)
                drain_write(1)

            ca0, cp0 = start_base(j0, 0)
            ca1, cp1 = start_base(j1, 1)
            ca0.wait()
            cb0 = start_add(j0, 0)
            ca1.wait()
            cb1 = start_add(j1, 1)
            chunk_back(j0, 0, cb0, cp0)
            chunk_back(j1, 1, cb1, cp1)
            return carry

        lax.fori_loop(0, nch // 2, group, 0)
        drain_write(0)
        drain_write(1)

    if with_pos:
        return k(tab_a, tab_b, pos_tab, idx_a, idx_b)
    return k(tab_a, tab_b, idx_a, idx_b)[0]


def _sc_scatter_add(scat0, scat1, idx2, zeros_hbm, n_rows):
    n_e = scat0.shape[0] * 2
    chs = 80
    per_w = n_e // NWORK
    nch = per_w // chs
    stripe = n_rows // 16
    nhalf = NWORK // 2
    mesh = plsc.VectorSubcoreMesh(core_axis_name="c", subcore_axis_name="s", num_cores=2, num_subcores=16)

    @functools.partial(
        pl.kernel,
        out_type=jax.ShapeDtypeStruct((2, n_rows, SCATW), jnp.float32),
        mesh=mesh,
        compiler_params=pltpu.CompilerParams(use_tc_tiling_on_sc=False),
        scratch_types=[
            pltpu.VMEM((nch, chs), jnp.int32),
            pltpu.VMEM((2, chs, SCATW), jnp.float32),
            pltpu.VMEM_SHARED((n_rows, SCATW), jnp.float32),
            pltpu.SemaphoreType.DMA,
            pltpu.SemaphoreType.DMA,
            pltpu.SemaphoreType.DMA,
            pltpu.SemaphoreType.DMA,
        ],
    )
    def k(s0_hbm, s1_hbm, di_hbm, z_hbm, out_hbm, di_v, rows_v, agg_sh,
          sr0, sr1, sa0, sa1):
        c = lax.axis_index("c")
        s = lax.axis_index("s")
        wid = s * 2 + c
        pltpu.sync_copy(z_hbm.at[pl.ds(s * stripe, stripe)],
                        agg_sh.at[pl.ds(s * stripe, stripe)])
        plsc.subcore_barrier()
        pltpu.sync_copy(di_hbm.at[pl.ds(wid * nch, nch)], di_v)
        sr = (sr0, sr1)
        sa = (sa0, sa1)

        def start_read(j, b):
            off = wid * per_w + j * chs

            @pl.when(wid < nhalf)
            def _():
                pltpu.async_copy(s0_hbm.at[pl.ds(off, chs)],
                                 rows_v.at[b], sr[b])

            @pl.when(wid >= nhalf)
            def _():
                pltpu.async_copy(
                    s1_hbm.at[pl.ds(off - nhalf * per_w, chs)],
                    rows_v.at[b], sr[b])

        def wait_read(b):
            pltpu.make_async_copy(s0_hbm.at[pl.ds(0, chs)], rows_v.at[b],
                                  sr[b]).wait()

        def drain_add(b):
            pltpu.make_async_copy(rows_v.at[b], agg_sh.at[di_v.at[0]],
                                  sa[b]).wait()

        def group(g, carry):
            j0 = g * 2
            j1 = j0 + 1

            @pl.when(g > 0)
            def _():
                drain_add(0)
                drain_add(1)

            start_read(j0, 0)
            start_read(j1, 1)
            wait_read(0)
            pltpu.async_copy(rows_v.at[0], agg_sh.at[di_v.at[j0]], sa[0],
                             add=True)
            wait_read(1)
            pltpu.async_copy(rows_v.at[1], agg_sh.at[di_v.at[j1]], sa[1],
                             add=True)
            return carry

        lax.fori_loop(0, nch // 2, group, 0)
        drain_add(0)
        drain_add(1)
        plsc.subcore_barrier()
        pltpu.sync_copy(agg_sh.at[pl.ds(s * stripe, stripe)],
                        out_hbm.at[c, pl.ds(s * stripe, stripe)])

    return k(scat0, scat1, idx2, zeros_hbm)


def _tc_init1(x, z, wx, wz, b8, bn):
    n = x.shape[0]

    def body(x_ref, z_ref, wx_ref, wz_ref, b_ref, s_ref, praw_ref):
        sp = jnp.dot(x_ref[...], wx_ref[...], preferred_element_type=jnp.float32)
        sp = sp + jnp.dot(z_ref[...], wz_ref[...], preferred_element_type=jnp.float32)
        sp = sp + b_ref[0:1, :]
        s_ref[...] = sp[:, :SDIM]
        praw_ref[...] = sp[:, SDIM:SDIM + 16]

    return pl.pallas_call(
        body,
        grid=(n // bn,),
        in_specs=[
            pl.BlockSpec((bn, x.shape[1]), lambda i: (i, 0)),
            pl.BlockSpec((bn, z.shape[1]), lambda i: (i, 0)),
            pl.BlockSpec(wx.shape, lambda i: (0, 0)),
            pl.BlockSpec(wz.shape, lambda i: (0, 0)),
            pl.BlockSpec(b8.shape, lambda i: (0, 0)),
        ],
        out_specs=[
            pl.BlockSpec((bn, SDIM), lambda i: (i, 0)),
            pl.BlockSpec((bn, 16), lambda i: (i, 0)),
        ],
        out_shape=[
            jax.ShapeDtypeStruct((n, SDIM), jnp.float32),
            jax.ShapeDtypeStruct((n, 16), jnp.float32),
        ],
    )(x, z, wx, wz, b8)


def _tc_init2(s0, praw, r16, wa, wb, bn):
    n = s0.shape[0]

    def body(s_ref, p_ref, r_ref, wa_ref, wb_ref, ta_ref, tb_ref, pos_ref):
        p = p_ref[...]
        r = r_ref[...]
        cols = []
        for i in range(3):
            acc = r[:, 3 * i:3 * i + 1] * p[:, 4:5]
            acc = acc + r[:, 3 * i + 1:3 * i + 2] * p[:, 5:6]
            acc = acc + r[:, 3 * i + 2:3 * i + 3] * p[:, 6:7]
            cols.append(acc)
        zc = jnp.zeros((bn, 4), jnp.float32)
        pos16 = jnp.concatenate(
            [zc, cols[0], cols[1], cols[2], jnp.zeros((bn, 9), jnp.float32)],
            axis=1)
        pos_ref[...] = pos16
        pad = jnp.concatenate([jnp.zeros((bn, SCATW), jnp.float32), pos16],
                              axis=1)
        sv = s_ref[...]
        ta_ref[...] = jnp.dot(sv, wa_ref[...], preferred_element_type=jnp.float32) + pad
        tb_ref[...] = jnp.dot(sv, wb_ref[...], preferred_element_type=jnp.float32) + pad

    return pl.pallas_call(
        body,
        grid=(n // bn,),
        in_specs=[
            pl.BlockSpec((bn, SDIM), lambda i: (i, 0)),
            pl.BlockSpec((bn, 16), lambda i: (i, 0)),
            pl.BlockSpec((bn, 16), lambda i: (i, 0)),
            pl.BlockSpec(wa.shape, lambda i: (0, 0)),
            pl.BlockSpec(wb.shape, lambda i: (0, 0)),
        ],
        out_specs=[
            pl.BlockSpec((bn, TW), lambda i: (i, 0)),
            pl.BlockSpec((bn, TW), lambda i: (i, 0)),
            pl.BlockSpec((bn, 16), lambda i: (i, 0)),
        ],
        out_shape=[
            jax.ShapeDtypeStruct((n, TW), jnp.float32),
            jax.ShapeDtypeStruct((n, TW), jnp.float32),
            jax.ShapeDtypeStruct((n, 16), jnp.float32),
        ],
    )(s0, praw, r16, wa, wb)


def _tc_edge(gsum, gps, e, wc, wdweb, be, e_init=None):
    n_e = gsum.shape[0]

    def body(gs_ref, gp_ref, e_ref, wc_ref, w3_ref, *rest):
        if e_init is not None:
            bw_ref, bb_ref, scat_ref, enew_ref = rest
        else:
            scat_ref, enew_ref = rest
        pmask = _posmask(16)
        cmask = _cntmask(16)
        gs = gs_ref[...]
        ps = gp_ref[...]
        if e_init is not None:
            ev = (jnp.dot(e_ref[...], bw_ref[...],
                          preferred_element_type=jnp.float32)
                  + bb_ref[0:1, :])
        else:
            ev = e_ref[...]
        pre = gs + jnp.dot(ev, wc_ref[...], preferred_element_type=jnp.float32)
        pd = gs[:, SCATW:TW] * pmask - ps
        r = pd - ps
        d2 = jnp.sum(r * r, axis=1, keepdims=True)
        d = jnp.sqrt(jnp.maximum(d2, 1e-6))
        a = jnp.sum(pd * ps, axis=1, keepdims=True)
        pre = pre + d * w3_ref[0:1, :] + a * w3_ref[1:2, :] + w3_ref[2:3, :]
        m = pre * jax.nn.sigmoid(pre)
        rn = r / (1.0 + d)
        scatblk = rn * m[:, KEEP - 1:KEEP] + cmask
        scat_ref[...] = jnp.concatenate([m[:, :SDIM], scatblk], axis=1)
        enew_ref[...] = ev + m[:, SDIM:SDIM + EDIM]

    in_specs = [
        pl.BlockSpec((be, TW), lambda i: (i, 0)),
        pl.BlockSpec((be, 16), lambda i: (i, 0)),
        pl.BlockSpec((be, e.shape[1]), lambda i: (i, 0)),
        pl.BlockSpec(wc.shape, lambda i: (0, 0)),
        pl.BlockSpec(wdweb.shape, lambda i: (0, 0)),
    ]
    args = [gsum, gps, e, wc, wdweb]
    if e_init is not None:
        bw8, bb16 = e_init
        in_specs += [pl.BlockSpec(bw8.shape, lambda i: (0, 0)),
                     pl.BlockSpec(bb16.shape, lambda i: (0, 0))]
        args += [bw8, bb16]
    return pl.pallas_call(
        body,
        grid=(n_e // be,),
        in_specs=in_specs,
        out_specs=[
            pl.BlockSpec((be, SCATW), lambda i: (i, 0)),
            pl.BlockSpec((be, EDIM), lambda i: (i, 0)),
        ],
        out_shape=[
            jax.ShapeDtypeStruct((n_e, SCATW), jnp.float32),
            jax.ShapeDtypeStruct((n_e, EDIM), jnp.float32),
        ],
    )(*args)


def _tc_node(agg2, s, pos16, ws, bs8, wan, wbn, bn):
    n = s.shape[0]

    def body(ag_ref, s_ref, p_ref, ws_ref, bs_ref, wa_ref, wb_ref,
             sn_ref, pn_ref, ta_ref, tb_ref):
        pmask = _posmask(16)
        ag = ag_ref[0] + ag_ref[1]
        denom = jnp.maximum(ag[:, SCATW - 1:SCATW], 1.0)
        aggs = ag[:, :SDIM] / denom
        u = (jnp.dot(aggs, ws_ref[...], preferred_element_type=jnp.float32)
             + bs_ref[0:1, :])
        sn = s_ref[...] + u * jax.nn.sigmoid(u)
        posn = p_ref[...] + (ag[:, SDIM:SCATW] * pmask) / denom
        sn_ref[...] = sn
        pn_ref[...] = posn
        pad = jnp.concatenate([jnp.zeros((bn, SCATW), jnp.float32), posn],
                              axis=1)
        ta_ref[...] = jnp.dot(sn, wa_ref[...], preferred_element_type=jnp.float32) + pad
        tb_ref[...] = jnp.dot(sn, wb_ref[...], preferred_element_type=jnp.float32) + pad

    return pl.pallas_call(
        body,
        grid=(n // bn,),
        in_specs=[
            pl.BlockSpec((2, bn, SCATW), lambda i: (0, i, 0)),
            pl.BlockSpec((bn, SDIM), lambda i: (i, 0)),
            pl.BlockSpec((bn, 16), lambda i: (i, 0)),
            pl.BlockSpec(ws.shape, lambda i: (0, 0)),
            pl.BlockSpec(bs8.shape, lambda i: (0, 0)),
            pl.BlockSpec(wan.shape, lambda i: (0, 0)),
            pl.BlockSpec(wbn.shape, lambda i: (0, 0)),
        ],
        out_specs=[
            pl.BlockSpec((bn, SDIM), lambda i: (i, 0)),
            pl.BlockSpec((bn, 16), lambda i: (i, 0)),
            pl.BlockSpec((bn, TW), lambda i: (i, 0)),
            pl.BlockSpec((bn, TW), lambda i: (i, 0)),
        ],
        out_shape=[
            jax.ShapeDtypeStruct((n, SDIM), jnp.float32),
            jax.ShapeDtypeStruct((n, 16), jnp.float32),
            jax.ShapeDtypeStruct((n, TW), jnp.float32),
            jax.ShapeDtypeStruct((n, TW), jnp.float32),
        ],
    )(agg2, s, pos16, ws, bs8, wan, wbn)


def _tc_head_node(agg2, s, pos16, ws, bs8, hwa, hba8, wbp, wbq, bn):
    n = s.shape[0]

    def body(ag_ref, s_ref, p_ref, ws_ref, bs_ref, hwa_ref, hba_ref,
             wp_ref, wq_ref, at_ref, pn_ref, tp_ref, tq_ref):
        pmask = _posmask(16)
        ag = ag_ref[0] + ag_ref[1]
        denom = jnp.maximum(ag[:, SCATW - 1:SCATW], 1.0)
        aggs = ag[:, :SDIM] / denom
        u = (jnp.dot(aggs, ws_ref[...], preferred_element_type=jnp.float32)
             + bs_ref[0:1, :])
        sn = s_ref[...] + u * jax.nn.sigmoid(u)
        pn_ref[...] = p_ref[...] + (ag[:, SDIM:SCATW] * pmask) / denom
        at_ref[...] = (jnp.dot(sn, hwa_ref[...],
                               preferred_element_type=jnp.float32)
                       + hba_ref[0:1, :])
        tp_ref[...] = jnp.dot(sn, wp_ref[...], preferred_element_type=jnp.float32)
        tq_ref[...] = jnp.dot(sn, wq_ref[...], preferred_element_type=jnp.float32)

    return pl.pallas_call(
        body,
        grid=(n // bn,),
        in_specs=[
            pl.BlockSpec((2, bn, SCATW), lambda i: (0, i, 0)),
            pl.BlockSpec((bn, SDIM), lambda i: (i, 0)),
            pl.BlockSpec((bn, 16), lambda i: (i, 0)),
            pl.BlockSpec(ws.shape, lambda i: (0, 0)),
            pl.BlockSpec(bs8.shape, lambda i: (0, 0)),
            pl.BlockSpec(hwa.shape, lambda i: (0, 0)),
            pl.BlockSpec(hba8.shape, lambda i: (0, 0)),
            pl.BlockSpec(wbp.shape, lambda i: (0, 0)),
            pl.BlockSpec(wbq.shape, lambda i: (0, 0)),
        ],
        out_specs=[
            pl.BlockSpec((bn, hwa.shape[1]), lambda i: (i, 0)),
            pl.BlockSpec((bn, 16), lambda i: (i, 0)),
            pl.BlockSpec((bn, 16), lambda i: (i, 0)),
            pl.BlockSpec((bn, 16), lambda i: (i, 0)),
        ],
        out_shape=[
            jax.ShapeDtypeStruct((n, hwa.shape[1]), jnp.float32),
            jax.ShapeDtypeStruct((n, 16), jnp.float32),
            jax.ShapeDtypeStruct((n, 16), jnp.float32),
            jax.ShapeDtypeStruct((n, 16), jnp.float32),
        ],
    )(agg2, s, pos16, ws, bs8, hwa, hba8, wbp, wbq)


def _tc_bonds(gpq, e, wbe, bb16, be):
    n_e = gpq.shape[0]

    def body(pq_ref, e_ref, w_ref, b_ref, o_ref):
        o_ref[...] = (pq_ref[...]
                      + jnp.dot(e_ref[...], w_ref[...],
                                preferred_element_type=jnp.float32)
                      + b_ref[0:1, :])

    return pl.pallas_call(
        body,
        grid=(n_e // be,),
        in_specs=[
            pl.BlockSpec((be, 16), lambda i: (i, 0)),
            pl.BlockSpec((be, EDIM), lambda i: (i, 0)),
            pl.BlockSpec(wbe.shape, lambda i: (0, 0)),
            pl.BlockSpec(bb16.shape, lambda i: (0, 0)),
        ],
        out_specs=[pl.BlockSpec((be, 16), lambda i: (i, 0))],
        out_shape=[jax.ShapeDtypeStruct((n_e, 16), jnp.float32)],
    )(gpq, e, wbe, bb16)[0]


def kernel(x, z, rot, edge_attr_global, atom_W, atom_b, bond_W, bond_b,
           msg_W, msg_b, upd_Ws, upd_bs, upd_Wv, head_Wa, head_ba,
           head_Wb, head_bb, edge_index_global, batch):
    n = x.shape[0]
    e_cnt = edge_index_global.shape[1]
    nl = msg_W.shape[0]
    naf = x.shape[1]
    latd = z.shape[1]
    nbt = edge_attr_global.shape[1]
    msg_out = msg_W.shape[2]

    bn = 1000
    be = 2048
    grain = NWORK * CH
    e_pad = ((e_cnt + grain - 1) // grain) * grain
    n_rows = ((n + 1 + 127) // 128) * 128
    n_gpad = ((n + grain - 1) // grain) * grain

    f32 = jnp.float32
    src = edge_index_global[0].astype(jnp.int32)
    dst = edge_index_global[1].astype(jnp.int32)

    perm = jnp.concatenate([
        jnp.arange(0, SDIM, dtype=jnp.int32),
        jnp.arange(msg_out - EDIM, msg_out, dtype=jnp.int32),
        jnp.array([SDIM + 32], dtype=jnp.int32),
    ])
    msgk = msg_W[:, :, perm]
    msgbk = msg_b[:, perm]

    def padc(a, w):
        return jnp.pad(a, ((0, 0),) * (a.ndim - 1) + ((0, w - a.shape[-1]),))

    wa_l = padc(msgk[:, :SDIM, :], TW)
    wb_l = padc(msgk[:, SDIM:2 * SDIM, :], TW)
    wc_l = padc(msgk[:, 2 * SDIM:2 * SDIM + EDIM, :], TW)
    w3_l = jnp.zeros((nl, 8, TW), f32)
    w3_l = w3_l.at[:, 0, :KEEP].set(msgk[:, 2 * SDIM + EDIM, :])
    w3_l = w3_l.at[:, 1, :KEEP].set(msgk[:, 2 * SDIM + EDIM + 1, :])
    w3_l = w3_l.at[:, 2, :KEEP].set(msgbk)

    aw = jnp.zeros((naf + latd, SDIM + 16), f32)
    aw = aw.at[:, :SDIM].set(atom_W[:, :SDIM])
    aw = aw.at[:, SDIM + 4:SDIM + 7].set(atom_W[:, SDIM:SDIM + 3])
    wx = aw[:naf]
    wz = aw[naf:]
    ab8 = jnp.zeros((8, SDIM + 16), f32)
    ab8 = ab8.at[0, :SDIM].set(atom_b[:SDIM])
    ab8 = ab8.at[0, SDIM + 4:SDIM + 7].set(atom_b[SDIM:SDIM + 3])

    bw8 = jnp.zeros((8, EDIM), f32).at[:nbt].set(bond_W)
    bb_16 = jnp.zeros((8, EDIM), f32).at[0].set(bond_b)

    bs8_l = jnp.zeros((nl, 8, SDIM), f32).at[:, 0].set(upd_bs)
    hba8 = jnp.zeros((8, naf), f32).at[0].set(head_ba)
    wbp = padc(head_Wb[:SDIM], 16)
    wbq = padc(head_Wb[SDIM:2 * SDIM], 16)
    wbe = padc(head_Wb[2 * SDIM:2 * SDIM + EDIM], 16)
    hbb16 = jnp.zeros((8, 16), f32).at[0, :nbt].set(head_bb)

    rot16 = padc(rot.reshape(rot.shape[0], 9), 16)

    half = e_pad // 2
    src_g = jnp.concatenate([src, jnp.zeros((e_pad - e_cnt,), jnp.int32)])
    dst_g = jnp.concatenate([dst, jnp.zeros((e_pad - e_cnt,), jnp.int32)])
    dst_sflat = jnp.concatenate(
        [dst, jnp.full((e_pad - e_cnt,), n, jnp.int32)])
    src_h = (src_g[:half], src_g[half:])
    dst_h = (dst_g[:half], dst_g[half:])
    dst_s2 = dst_sflat.reshape(-1, 80)
    batch_g = jnp.concatenate(
        [batch.astype(jnp.int32), jnp.zeros((n_gpad - n,), jnp.int32)])

    ea8 = jnp.zeros((e_pad, 8), f32).at[:e_cnt, :nbt].set(edge_attr_global)
    ea8_h = (ea8[:half], ea8[half:])
    zeros_acc = jnp.zeros((n_rows, SCATW), f32)

    s0, praw = _tc_init1(x, z, wx, wz, ab8, bn)
    r16g, _ = _sc_gather2(rot16, rot16, batch_g, batch_g, n_gpad)
    ta, tb, pos16 = _tc_init2(s0, praw, r16g[:n], wa_l[0], wb_l[0], bn)

    s = s0
    e_h = [None, None]
    for l in range(nl):
        scat_h = []
        for h in range(2):
            gsum, gps = _sc_gather_sum(ta, tb, pos16, src_h[h], dst_h[h],
                                       half)
            if l == 0:
                scat, enew = _tc_edge(gsum, gps, ea8_h[h], wc_l[l], w3_l[l],
                                      be, e_init=(bw8, bb_16))
            else:
                scat, enew = _tc_edge(gsum, gps, e_h[h], wc_l[l], w3_l[l], be)
            e_h[h] = enew
            scat_h.append(scat)
        agg2 = _sc_scatter_add(scat_h[0], scat_h[1], dst_s2, zeros_acc,
                               n_rows)
        if l + 1 < nl:
            s, pos16, ta, tb = _tc_node(agg2, s, pos16, upd_Ws[l], bs8_l[l],
                                        wa_l[l + 1], wb_l[l + 1], bn)
        else:
            atoms, pos16, tp, tq = _tc_head_node(
                agg2, s, pos16, upd_Ws[l], bs8_l[l], head_Wa, hba8,
                wbp, wbq, bn)

    gpq = _sc_gather_sum(tp, tq, None, src_g, dst_g, e_pad)
    bonds_h = []
    for h in range(2):
        bonds_h.append(_tc_bonds(gpq[h * half:(h + 1) * half], e_h[h],
                                 wbe, hbb16, be))

    bonds = jnp.concatenate(bonds_h)[:e_cnt, :nbt]
    pos = pos16[:, 4:7]
    return atoms, bonds, pos


# speedup vs baseline: 1.0233x; 1.0233x over previous
import functools

import jax
import jax.numpy as jnp
from jax import lax
from jax.experimental import pallas as pl
from jax.experimental.pallas import tpu as pltpu
from jax.experimental.pallas import tpu_sc as plsc

SDIM = 128
EDIM = 16
KEEP = 145
TW = 160
SCATW = 144
CH = 128
NWORK = 32


def _posmask(w):
    io = lax.broadcasted_iota(jnp.int32, (1, w), 1)
    return jnp.where((io >= 4) & (io < 7), 1.0, 0.0).astype(jnp.float32)


def _cntmask(w):
    io = lax.broadcasted_iota(jnp.int32, (1, w), 1)
    return jnp.where(io == (w - 1), 1.0, 0.0).astype(jnp.float32)


def _sc_gather2(tab_a, tab_b, idx_a, idx_b, n_out):
    wa = tab_a.shape[1]
    wb = tab_b.shape[1]
    per_w = n_out // NWORK
    nch = per_w // CH
    mesh = plsc.VectorSubcoreMesh(core_axis_name="c", subcore_axis_name="s", num_cores=2, num_subcores=16)

    @functools.partial(
        pl.kernel,
        out_type=(jax.ShapeDtypeStruct((n_out, wa), jnp.float32),
                  jax.ShapeDtypeStruct((n_out, wb), jnp.float32)),
        mesh=mesh,
        compiler_params=pltpu.CompilerParams(use_tc_tiling_on_sc=False),
        scratch_types=[
            pltpu.VMEM((per_w,), jnp.int32),
            pltpu.VMEM((per_w,), jnp.int32),
            pltpu.VMEM((2, CH, wa), jnp.float32),
            pltpu.VMEM((2, CH, wb), jnp.float32),
            pltpu.SemaphoreType.DMA,
            pltpu.SemaphoreType.DMA,
            pltpu.SemaphoreType.DMA,
            pltpu.SemaphoreType.DMA,
            pltpu.SemaphoreType.DMA,
            pltpu.SemaphoreType.DMA,
            pltpu.SemaphoreType.DMA,
            pltpu.SemaphoreType.DMA,
        ],
    )
    def k(ta_hbm, tb_hbm, ia_hbm, ib_hbm, ga_hbm, gb_hbm,
          ia_v, ib_v, ra_v, rb_v,
          sga0, sga1, sgb0, sgb1, swa0, swa1, swb0, swb1):
        wid = lax.axis_index("s") * 2 + lax.axis_index("c")
        base = wid * per_w
        pltpu.sync_copy(ia_hbm.at[pl.ds(base, per_w)], ia_v)
        pltpu.sync_copy(ib_hbm.at[pl.ds(base, per_w)], ib_v)
        sga = (sga0, sga1)
        sgb = (sgb0, sgb1)
        swa = (swa0, swa1)
        swb = (swb0, swb1)

        def start_gather(j, b):
            ca = pltpu.async_copy(ta_hbm.at[ia_v.at[pl.ds(j * CH, CH)]],
                                  ra_v.at[b], sga[b])
            cb = pltpu.async_copy(tb_hbm.at[ib_v.at[pl.ds(j * CH, CH)]],
                                  rb_v.at[b], sgb[b])
            return ca, cb

        def start_write(j, b):
            pltpu.async_copy(ra_v.at[b], ga_hbm.at[pl.ds(base + j * CH, CH)],
                             swa[b])
            pltpu.async_copy(rb_v.at[b], gb_hbm.at[pl.ds(base + j * CH, CH)],
                             swb[b])

        def drain_write(b):
            pltpu.make_async_copy(ra_v.at[b], ga_hbm.at[pl.ds(base, CH)],
                                  swa[b]).wait()
            pltpu.make_async_copy(rb_v.at[b], gb_hbm.at[pl.ds(base, CH)],
                                  swb[b]).wait()

        def group(g, carry):
            j0 = g * 2
            j1 = j0 + 1

            @pl.when(g > 0)
            def _():
                drain_write(0)
                drain_write(1)

            ca0, cb0 = start_gather(j0, 0)
            ca1, cb1 = start_gather(j1, 1)
            ca0.wait()
            cb0.wait()
            start_write(j0, 0)
            ca1.wait()
            cb1.wait()
            start_write(j1, 1)
            return carry

        lax.fori_loop(0, nch // 2, group, 0)
        if nch % 2:
            j = nch - 1
            drain_write(0)
            ca, cb = start_gather(j, 0)
            ca.wait()
            cb.wait()
            start_write(j, 0)
            drain_write(0)
            drain_write(1)
        else:
            drain_write(0)
            drain_write(1)

    return k(tab_a, tab_b, idx_a, idx_b)


def _sc_gather_sum(tab_a, tab_b, pos_tab, idx_a, idx_b, n_out):
    w = tab_a.shape[1]
    per_w = n_out // NWORK
    nch = per_w // CH
    assert nch % 2 == 0
    mesh = plsc.VectorSubcoreMesh(core_axis_name="c", subcore_axis_name="s", num_cores=2, num_subcores=16)
    with_pos = pos_tab is not None

    out_type = [jax.ShapeDtypeStruct((n_out, w), jnp.float32)]
    scratch = [
        pltpu.VMEM((per_w,), jnp.int32),
        pltpu.VMEM((per_w,), jnp.int32),
        pltpu.VMEM((2, CH, w), jnp.float32),
    ]
    if with_pos:
        out_type.append(jax.ShapeDtypeStruct((n_out, 16), jnp.float32))
        scratch.append(pltpu.VMEM((2, CH, 16), jnp.float32))
    scratch += [pltpu.SemaphoreType.DMA] * (10 if with_pos else 6)

    @functools.partial(
        pl.kernel,
        out_type=tuple(out_type),
        mesh=mesh,
        compiler_params=pltpu.CompilerParams(use_tc_tiling_on_sc=False),
        scratch_types=scratch,
    )
    def k(*refs):
        if with_pos:
            (ta_hbm, tb_hbm, pt_hbm, ia_hbm, ib_hbm, gs_hbm, gp_hbm,
             ia_v, ib_v, ra_v, rp_v,
             sga0, sga1, sgb0, sgb1, sgp0, sgp1, swa0, swa1, swp0, swp1) = refs
            sgp = (sgp0, sgp1)
            swp = (swp0, swp1)
        else:
            (ta_hbm, tb_hbm, ia_hbm, ib_hbm, gs_hbm,
             ia_v, ib_v, ra_v,
             sga0, sga1, sgb0, sgb1, swa0, swa1) = refs
        sga = (sga0, sga1)
        sgb = (sgb0, sgb1)
        swa = (swa0, swa1)
        wid = lax.axis_index("s") * 2 + lax.axis_index("c")
        base = wid * per_w
        pltpu.sync_copy(ia_hbm.at[pl.ds(base, per_w)], ia_v)
        pltpu.sync_copy(ib_hbm.at[pl.ds(base, per_w)], ib_v)

        def start_base(j, b):
            ca = pltpu.async_copy(ta_hbm.at[ia_v.at[pl.ds(j * CH, CH)]],
                                  ra_v.at[b], sga[b])
            cp = None
            if with_pos:
                cp = pltpu.async_copy(pt_hbm.at[ia_v.at[pl.ds(j * CH, CH)]],
                                      rp_v.at[b], sgp[b])
            return ca, cp

        def start_add(j, b):
            return pltpu.async_copy(tb_hbm.at[ib_v.at[pl.ds(j * CH, CH)]],
                                    ra_v.at[b], sgb[b], add=True)

        def chunk_back(j, b, cb, cp):
            cb.wait()
            pltpu.async_copy(ra_v.at[b], gs_hbm.at[pl.ds(base + j * CH, CH)],
                             swa[b])
            if with_pos:
                cp.wait()
                pltpu.async_copy(rp_v.at[b],
                                 gp_hbm.at[pl.ds(base + j * CH, CH)], swp[b])

        def drain_write(b):
            pltpu.make_async_copy(ra_v.at[b], gs_hbm.at[pl.ds(base, CH)],
                                  swa[b]).wait()
            if with_pos:
                pltpu.make_async_copy(rp_v.at[b], gp_hbm.at[pl.ds(base, CH)],
                                      swp[b]).wait()

        def group(g, carry):
            j0 = g * 2
            j1 = j0 + 1

            @pl.when(g > 0)
            def _():
                drain_write(0)
                drain_write(1)

            ca0, cp0 = start_base(j0, 0)
            ca0.wait()
            cb0 = start_add(j0, 0)
            ca1, cp1 = start_base(j1, 1)
            ca1.wait()
            cb1 = start_add(j1, 1)
            chunk_back(j0, 0, cb0, cp0)
            chunk_back(j1, 1, cb1, cp1)
            return carry

        lax.fori_loop(0, nch // 2, group, 0)
        drain_write(0)
        drain_write(1)

    if with_pos:
        return k(tab_a, tab_b, pos_tab, idx_a, idx_b)
    return k(tab_a, tab_b, idx_a, idx_b)[0]


def _sc_scatter_add(scat0, scat1, idx2, zeros_hbm, n_rows):
    n_e = scat0.shape[0] * 2
    chs = 80
    per_w = n_e // NWORK
    nch = per_w // chs
    stripe = n_rows // 16
    nhalf = NWORK // 2
    mesh = plsc.VectorSubcoreMesh(core_axis_name="c", subcore_axis_name="s", num_cores=2, num_subcores=16)

    @functools.partial(
        pl.kernel,
        out_type=jax.ShapeDtypeStruct((2, n_rows, SCATW), jnp.float32),
        mesh=mesh,
        compiler_params=pltpu.CompilerParams(use_tc_tiling_on_sc=False),
        scratch_types=[
            pltpu.VMEM((nch, chs), jnp.int32),
            pltpu.VMEM((2, chs, SCATW), jnp.float32),
            pltpu.VMEM_SHARED((n_rows, SCATW), jnp.float32),
            pltpu.SemaphoreType.DMA,
            pltpu.SemaphoreType.DMA,
            pltpu.SemaphoreType.DMA,
            pltpu.SemaphoreType.DMA,
        ],
    )
    def k(s0_hbm, s1_hbm, di_hbm, z_hbm, out_hbm, di_v, rows_v, agg_sh,
          sr0, sr1, sa0, sa1):
        c = lax.axis_index("c")
        s = lax.axis_index("s")
        wid = s * 2 + c
        pltpu.sync_copy(z_hbm.at[pl.ds(s * stripe, stripe)],
                        agg_sh.at[pl.ds(s * stripe, stripe)])
        plsc.subcore_barrier()
        pltpu.sync_copy(di_hbm.at[pl.ds(wid * nch, nch)], di_v)
        sr = (sr0, sr1)
        sa = (sa0, sa1)

        def start_read(j, b):
            off = wid * per_w + j * chs

            @pl.when(wid < nhalf)
            def _():
                pltpu.async_copy(s0_hbm.at[pl.ds(off, chs)],
                                 rows_v.at[b], sr[b])

            @pl.when(wid >= nhalf)
            def _():
                pltpu.async_copy(
                    s1_hbm.at[pl.ds(off - nhalf * per_w, chs)],
                    rows_v.at[b], sr[b])

        def wait_read(b):
            pltpu.make_async_copy(s0_hbm.at[pl.ds(0, chs)], rows_v.at[b],
                                  sr[b]).wait()

        def drain_add(b):
            pltpu.make_async_copy(rows_v.at[b], agg_sh.at[di_v.at[0]],
                                  sa[b]).wait()

        def group(g, carry):
            j0 = g * 2
            j1 = j0 + 1

            @pl.when(g > 0)
            def _():
                drain_add(0)
                drain_add(1)

            start_read(j0, 0)
            start_read(j1, 1)
            wait_read(0)
            pltpu.async_copy(rows_v.at[0], agg_sh.at[di_v.at[j0]], sa[0],
                             add=True)
            wait_read(1)
            pltpu.async_copy(rows_v.at[1], agg_sh.at[di_v.at[j1]], sa[1],
                             add=True)
            return carry

        lax.fori_loop(0, nch // 2, group, 0)
        drain_add(0)
        drain_add(1)
        plsc.subcore_barrier()
        pltpu.sync_copy(agg_sh.at[pl.ds(s * stripe, stripe)],
                        out_hbm.at[c, pl.ds(s * stripe, stripe)])

    return k(scat0, scat1, idx2, zeros_hbm)


def _tc_init1(x, z, wx, wz, b8, bn):
    n = x.shape[0]

    def body(x_ref, z_ref, wx_ref, wz_ref, b_ref, s_ref, praw_ref):
        sp = jnp.dot(x_ref[...], wx_ref[...], preferred_element_type=jnp.float32)
        sp = sp + jnp.dot(z_ref[...], wz_ref[...], preferred_element_type=jnp.float32)
        sp = sp + b_ref[0:1, :]
        s_ref[...] = sp[:, :SDIM]
        praw_ref[...] = sp[:, SDIM:SDIM + 16]

    return pl.pallas_call(
        body,
        grid=(n // bn,),
        in_specs=[
            pl.BlockSpec((bn, x.shape[1]), lambda i: (i, 0)),
            pl.BlockSpec((bn, z.shape[1]), lambda i: (i, 0)),
            pl.BlockSpec(wx.shape, lambda i: (0, 0)),
            pl.BlockSpec(wz.shape, lambda i: (0, 0)),
            pl.BlockSpec(b8.shape, lambda i: (0, 0)),
        ],
        out_specs=[
            pl.BlockSpec((bn, SDIM), lambda i: (i, 0)),
            pl.BlockSpec((bn, 16), lambda i: (i, 0)),
        ],
        out_shape=[
            jax.ShapeDtypeStruct((n, SDIM), jnp.float32),
            jax.ShapeDtypeStruct((n, 16), jnp.float32),
        ],
    )(x, z, wx, wz, b8)


def _tc_init2(s0, praw, r16, wa, wb, bn):
    n = s0.shape[0]

    def body(s_ref, p_ref, r_ref, wa_ref, wb_ref, ta_ref, tb_ref, pos_ref):
        p = p_ref[...]
        r = r_ref[...]
        cols = []
        for i in range(3):
            acc = r[:, 3 * i:3 * i + 1] * p[:, 4:5]
            acc = acc + r[:, 3 * i + 1:3 * i + 2] * p[:, 5:6]
            acc = acc + r[:, 3 * i + 2:3 * i + 3] * p[:, 6:7]
            cols.append(acc)
        zc = jnp.zeros((bn, 4), jnp.float32)
        pos16 = jnp.concatenate(
            [zc, cols[0], cols[1], cols[2], jnp.zeros((bn, 9), jnp.float32)],
            axis=1)
        pos_ref[...] = pos16
        pad = jnp.concatenate([jnp.zeros((bn, SCATW), jnp.float32), pos16],
                              axis=1)
        sv = s_ref[...]
        ta_ref[...] = jnp.dot(sv, wa_ref[...], preferred_element_type=jnp.float32) + pad
        tb_ref[...] = jnp.dot(sv, wb_ref[...], preferred_element_type=jnp.float32) + pad

    return pl.pallas_call(
        body,
        grid=(n // bn,),
        in_specs=[
            pl.BlockSpec((bn, SDIM), lambda i: (i, 0)),
            pl.BlockSpec((bn, 16), lambda i: (i, 0)),
            pl.BlockSpec((bn, 16), lambda i: (i, 0)),
            pl.BlockSpec(wa.shape, lambda i: (0, 0)),
            pl.BlockSpec(wb.shape, lambda i: (0, 0)),
        ],
        out_specs=[
            pl.BlockSpec((bn, TW), lambda i: (i, 0)),
            pl.BlockSpec((bn, TW), lambda i: (i, 0)),
            pl.BlockSpec((bn, 16), lambda i: (i, 0)),
        ],
        out_shape=[
            jax.ShapeDtypeStruct((n, TW), jnp.float32),
            jax.ShapeDtypeStruct((n, TW), jnp.float32),
            jax.ShapeDtypeStruct((n, 16), jnp.float32),
        ],
    )(s0, praw, r16, wa, wb)


def _tc_edge(gsum, gps, e, wc, wdweb, be, e_init=None):
    n_e = gsum.shape[0]

    def body(gs_ref, gp_ref, e_ref, wc_ref, w3_ref, *rest):
        if e_init is not None:
            bw_ref, bb_ref, scat_ref, enew_ref = rest
        else:
            scat_ref, enew_ref = rest
        pmask = _posmask(16)
        cmask = _cntmask(16)
        gs = gs_ref[...]
        ps = gp_ref[...]
        if e_init is not None:
            ev = (jnp.dot(e_ref[...], bw_ref[...],
                          preferred_element_type=jnp.float32)
                  + bb_ref[0:1, :])
        else:
            ev = e_ref[...]
        pre = gs + jnp.dot(ev, wc_ref[...], preferred_element_type=jnp.float32)
        pd = gs[:, SCATW:TW] * pmask - ps
        r = pd - ps
        d2 = jnp.sum(r * r, axis=1, keepdims=True)
        d = jnp.sqrt(jnp.maximum(d2, 1e-6))
        a = jnp.sum(pd * ps, axis=1, keepdims=True)
        pre = pre + d * w3_ref[0:1, :] + a * w3_ref[1:2, :] + w3_ref[2:3, :]
        m = pre * jax.nn.sigmoid(pre)
        rn = r / (1.0 + d)
        scatblk = rn * m[:, KEEP - 1:KEEP] + cmask
        scat_ref[...] = jnp.concatenate([m[:, :SDIM], scatblk], axis=1)
        enew_ref[...] = ev + m[:, SDIM:SDIM + EDIM]

    in_specs = [
        pl.BlockSpec((be, TW), lambda i: (i, 0)),
        pl.BlockSpec((be, 16), lambda i: (i, 0)),
        pl.BlockSpec((be, e.shape[1]), lambda i: (i, 0)),
        pl.BlockSpec(wc.shape, lambda i: (0, 0)),
        pl.BlockSpec(wdweb.shape, lambda i: (0, 0)),
    ]
    args = [gsum, gps, e, wc, wdweb]
    if e_init is not None:
        bw8, bb16 = e_init
        in_specs += [pl.BlockSpec(bw8.shape, lambda i: (0, 0)),
                     pl.BlockSpec(bb16.shape, lambda i: (0, 0))]
        args += [bw8, bb16]
    return pl.pallas_call(
        body,
        grid=(n_e // be,),
        in_specs=in_specs,
        out_specs=[
            pl.BlockSpec((be, SCATW), lambda i: (i, 0)),
            pl.BlockSpec((be, EDIM), lambda i: (i, 0)),
        ],
        out_shape=[
            jax.ShapeDtypeStruct((n_e, SCATW), jnp.float32),
            jax.ShapeDtypeStruct((n_e, EDIM), jnp.float32),
        ],
    )(*args)


def _tc_node(agg2, s, pos16, ws, bs8, wan, wbn, bn):
    n = s.shape[0]

    def body(ag_ref, s_ref, p_ref, ws_ref, bs_ref, wa_ref, wb_ref,
             sn_ref, pn_ref, ta_ref, tb_ref):
        pmask = _posmask(16)
        ag = ag_ref[0] + ag_ref[1]
        denom = jnp.maximum(ag[:, SCATW - 1:SCATW], 1.0)
        aggs = ag[:, :SDIM] / denom
        u = (jnp.dot(aggs, ws_ref[...], preferred_element_type=jnp.float32)
             + bs_ref[0:1, :])
        sn = s_ref[...] + u * jax.nn.sigmoid(u)
        posn = p_ref[...] + (ag[:, SDIM:SCATW] * pmask) / denom
        sn_ref[...] = sn
        pn_ref[...] = posn
        pad = jnp.concatenate([jnp.zeros((bn, SCATW), jnp.float32), posn],
                              axis=1)
        ta_ref[...] = jnp.dot(sn, wa_ref[...], preferred_element_type=jnp.float32) + pad
        tb_ref[...] = jnp.dot(sn, wb_ref[...], preferred_element_type=jnp.float32) + pad

    return pl.pallas_call(
        body,
        grid=(n // bn,),
        in_specs=[
            pl.BlockSpec((2, bn, SCATW), lambda i: (0, i, 0)),
            pl.BlockSpec((bn, SDIM), lambda i: (i, 0)),
            pl.BlockSpec((bn, 16), lambda i: (i, 0)),
            pl.BlockSpec(ws.shape, lambda i: (0, 0)),
            pl.BlockSpec(bs8.shape, lambda i: (0, 0)),
            pl.BlockSpec(wan.shape, lambda i: (0, 0)),
            pl.BlockSpec(wbn.shape, lambda i: (0, 0)),
        ],
        out_specs=[
            pl.BlockSpec((bn, SDIM), lambda i: (i, 0)),
            pl.BlockSpec((bn, 16), lambda i: (i, 0)),
            pl.BlockSpec((bn, TW), lambda i: (i, 0)),
            pl.BlockSpec((bn, TW), lambda i: (i, 0)),
        ],
        out_shape=[
            jax.ShapeDtypeStruct((n, SDIM), jnp.float32),
            jax.ShapeDtypeStruct((n, 16), jnp.float32),
            jax.ShapeDtypeStruct((n, TW), jnp.float32),
            jax.ShapeDtypeStruct((n, TW), jnp.float32),
        ],
    )(agg2, s, pos16, ws, bs8, wan, wbn)


def _tc_head_node(agg2, s, pos16, ws, bs8, hwa, hba8, wbp, wbq, bn):
    n = s.shape[0]

    def body(ag_ref, s_ref, p_ref, ws_ref, bs_ref, hwa_ref, hba_ref,
             wp_ref, wq_ref, at_ref, pn_ref, tp_ref, tq_ref):
        pmask = _posmask(16)
        ag = ag_ref[0] + ag_ref[1]
        denom = jnp.maximum(ag[:, SCATW - 1:SCATW], 1.0)
        aggs = ag[:, :SDIM] / denom
        u = (jnp.dot(aggs, ws_ref[...], preferred_element_type=jnp.float32)
             + bs_ref[0:1, :])
        sn = s_ref[...] + u * jax.nn.sigmoid(u)
        pn_ref[...] = p_ref[...] + (ag[:, SDIM:SCATW] * pmask) / denom
        at_ref[...] = (jnp.dot(sn, hwa_ref[...],
                               preferred_element_type=jnp.float32)
                       + hba_ref[0:1, :])
        tp_ref[...] = jnp.dot(sn, wp_ref[...], preferred_element_type=jnp.float32)
        tq_ref[...] = jnp.dot(sn, wq_ref[...], preferred_element_type=jnp.float32)

    return pl.pallas_call(
        body,
        grid=(n // bn,),
        in_specs=[
            pl.BlockSpec((2, bn, SCATW), lambda i: (0, i, 0)),
            pl.BlockSpec((bn, SDIM), lambda i: (i, 0)),
            pl.BlockSpec((bn, 16), lambda i: (i, 0)),
            pl.BlockSpec(ws.shape, lambda i: (0, 0)),
            pl.BlockSpec(bs8.shape, lambda i: (0, 0)),
            pl.BlockSpec(hwa.shape, lambda i: (0, 0)),
            pl.BlockSpec(hba8.shape, lambda i: (0, 0)),
            pl.BlockSpec(wbp.shape, lambda i: (0, 0)),
            pl.BlockSpec(wbq.shape, lambda i: (0, 0)),
        ],
        out_specs=[
            pl.BlockSpec((bn, hwa.shape[1]), lambda i: (i, 0)),
            pl.BlockSpec((bn, 16), lambda i: (i, 0)),
            pl.BlockSpec((bn, 16), lambda i: (i, 0)),
            pl.BlockSpec((bn, 16), lambda i: (i, 0)),
        ],
        out_shape=[
            jax.ShapeDtypeStruct((n, hwa.shape[1]), jnp.float32),
            jax.ShapeDtypeStruct((n, 16), jnp.float32),
            jax.ShapeDtypeStruct((n, 16), jnp.float32),
            jax.ShapeDtypeStruct((n, 16), jnp.float32),
        ],
    )(agg2, s, pos16, ws, bs8, hwa, hba8, wbp, wbq)


def _tc_bonds(gpq, e, wbe, bb16, be):
    n_e = gpq.shape[0]

    def body(pq_ref, e_ref, w_ref, b_ref, o_ref):
        o_ref[...] = (pq_ref[...]
                      + jnp.dot(e_ref[...], w_ref[...],
                                preferred_element_type=jnp.float32)
                      + b_ref[0:1, :])

    return pl.pallas_call(
        body,
        grid=(n_e // be,),
        in_specs=[
            pl.BlockSpec((be, 16), lambda i: (i, 0)),
            pl.BlockSpec((be, EDIM), lambda i: (i, 0)),
            pl.BlockSpec(wbe.shape, lambda i: (0, 0)),
            pl.BlockSpec(bb16.shape, lambda i: (0, 0)),
        ],
        out_specs=[pl.BlockSpec((be, 16), lambda i: (i, 0))],
        out_shape=[jax.ShapeDtypeStruct((n_e, 16), jnp.float32)],
    )(gpq, e, wbe, bb16)[0]


def kernel(x, z, rot, edge_attr_global, atom_W, atom_b, bond_W, bond_b,
           msg_W, msg_b, upd_Ws, upd_bs, upd_Wv, head_Wa, head_ba,
           head_Wb, head_bb, edge_index_global, batch):
    n = x.shape[0]
    e_cnt = edge_index_global.shape[1]
    nl = msg_W.shape[0]
    naf = x.shape[1]
    latd = z.shape[1]
    nbt = edge_attr_global.shape[1]
    msg_out = msg_W.shape[2]

    bn = 1000
    be = 2048
    grain = NWORK * CH
    e_pad = ((e_cnt + grain - 1) // grain) * grain
    n_rows = ((n + 1 + 127) // 128) * 128
    n_gpad = ((n + grain - 1) // grain) * grain

    f32 = jnp.float32
    src = edge_index_global[0].astype(jnp.int32)
    dst = edge_index_global[1].astype(jnp.int32)

    perm = jnp.concatenate([
        jnp.arange(0, SDIM, dtype=jnp.int32),
        jnp.arange(msg_out - EDIM, msg_out, dtype=jnp.int32),
        jnp.array([SDIM + 32], dtype=jnp.int32),
    ])
    msgk = msg_W[:, :, perm]
    msgbk = msg_b[:, perm]

    def padc(a, w):
        return jnp.pad(a, ((0, 0),) * (a.ndim - 1) + ((0, w - a.shape[-1]),))

    wa_l = padc(msgk[:, :SDIM, :], TW)
    wb_l = padc(msgk[:, SDIM:2 * SDIM, :], TW)
    wc_l = padc(msgk[:, 2 * SDIM:2 * SDIM + EDIM, :], TW)
    w3_l = jnp.zeros((nl, 8, TW), f32)
    w3_l = w3_l.at[:, 0, :KEEP].set(msgk[:, 2 * SDIM + EDIM, :])
    w3_l = w3_l.at[:, 1, :KEEP].set(msgk[:, 2 * SDIM + EDIM + 1, :])
    w3_l = w3_l.at[:, 2, :KEEP].set(msgbk)

    aw = jnp.zeros((naf + latd, SDIM + 16), f32)
    aw = aw.at[:, :SDIM].set(atom_W[:, :SDIM])
    aw = aw.at[:, SDIM + 4:SDIM + 7].set(atom_W[:, SDIM:SDIM + 3])
    wx = aw[:naf]
    wz = aw[naf:]
    ab8 = jnp.zeros((8, SDIM + 16), f32)
    ab8 = ab8.at[0, :SDIM].set(atom_b[:SDIM])
    ab8 = ab8.at[0, SDIM + 4:SDIM + 7].set(atom_b[SDIM:SDIM + 3])

    bw8 = jnp.zeros((8, EDIM), f32).at[:nbt].set(bond_W)
    bb_16 = jnp.zeros((8, EDIM), f32).at[0].set(bond_b)

    bs8_l = jnp.zeros((nl, 8, SDIM), f32).at[:, 0].set(upd_bs)
    hba8 = jnp.zeros((8, naf), f32).at[0].set(head_ba)
    wbp = padc(head_Wb[:SDIM], 16)
    wbq = padc(head_Wb[SDIM:2 * SDIM], 16)
    wbe = padc(head_Wb[2 * SDIM:2 * SDIM + EDIM], 16)
    hbb16 = jnp.zeros((8, 16), f32).at[0, :nbt].set(head_bb)

    rot16 = padc(rot.reshape(rot.shape[0], 9), 16)

    half = e_pad // 2
    src_g = jnp.concatenate([src, jnp.zeros((e_pad - e_cnt,), jnp.int32)])
    dst_g = jnp.concatenate([dst, jnp.zeros((e_pad - e_cnt,), jnp.int32)])
    dst_sflat = jnp.concatenate(
        [dst, jnp.full((e_pad - e_cnt,), n, jnp.int32)])
    src_h = (src_g[:half], src_g[half:])
    dst_h = (dst_g[:half], dst_g[half:])
    dst_s2 = dst_sflat.reshape(-1, 80)
    batch_g = jnp.concatenate(
        [batch.astype(jnp.int32), jnp.zeros((n_gpad - n,), jnp.int32)])

    ea8 = jnp.zeros((e_pad, 8), f32).at[:e_cnt, :nbt].set(edge_attr_global)
    ea8_h = (ea8[:half], ea8[half:])
    zeros_acc = jnp.zeros((n_rows, SCATW), f32)

    s0, praw = _tc_init1(x, z, wx, wz, ab8, bn)
    r16g, _ = _sc_gather2(rot16, rot16, batch_g, batch_g, n_gpad)
    ta, tb, pos16 = _tc_init2(s0, praw, r16g[:n], wa_l[0], wb_l[0], bn)

    s = s0
    e_h = [None, None]
    for l in range(nl):
        scat_h = []
        for h in range(2):
            gsum, gps = _sc_gather_sum(ta, tb, pos16, src_h[h], dst_h[h],
                                       half)
            if l == 0:
                scat, enew = _tc_edge(gsum, gps, ea8_h[h], wc_l[l], w3_l[l],
                                      be, e_init=(bw8, bb_16))
            else:
                scat, enew = _tc_edge(gsum, gps, e_h[h], wc_l[l], w3_l[l], be)
            e_h[h] = enew
            scat_h.append(scat)
        agg2 = _sc_scatter_add(scat_h[0], scat_h[1], dst_s2, zeros_acc,
                               n_rows)
        if l + 1 < nl:
            s, pos16, ta, tb = _tc_node(agg2, s, pos16, upd_Ws[l], bs8_l[l],
                                        wa_l[l + 1], wb_l[l + 1], bn)
        else:
            atoms, pos16, tp, tq = _tc_head_node(
                agg2, s, pos16, upd_Ws[l], bs8_l[l], head_Wa, hba8,
                wbp, wbq, bn)

    gpq = _sc_gather_sum(tp, tq, None, src_g, dst_g, e_pad)
    bonds_h = []
    for h in range(2):
        bonds_h.append(_tc_bonds(gpq[h * half:(h + 1) * half], e_h[h],
                                 wbe, hbb16, be))

    bonds = jnp.concatenate(bonds_h)[:e_cnt, :nbt]
    pos = pos16[:, 4:7]
    return atoms, bonds, pos
